# baseline (speedup 1.0000x reference)
import functools

import jax
import jax.numpy as jnp
from jax import lax
from jax.experimental import pallas as pl
from jax.experimental.pallas import tpu as pltpu
from jax.experimental.pallas import tpu_sc as plsc

D_MODEL = 64
SCALE = 8.0
LANES = 16

NUM_CORES = 2
NUM_SUBCORES = 16
NUM_WORKERS = NUM_CORES * NUM_SUBCORES

NB_TOTAL = 4096
SEQ = 50
NB_PER_WORKER = NB_TOTAL // NUM_WORKERS
NB_CHUNK = 8
N_CHUNKS = NB_PER_WORKER // NB_CHUNK
GROUPS = [(0, 0, 16), (16, 0, 16), (32, 0, 16), (34, 14, 16)]


def _make_gather_kernel():
    mesh = plsc.VectorSubcoreMesh(core_axis_name="c", subcore_axis_name="s")

    @functools.partial(
        pl.kernel,
        mesh=mesh,
        out_type=jax.ShapeDtypeStruct((NB_TOTAL, SEQ, D_MODEL), jnp.float32),
        scratch_types=[
            pltpu.VMEM((NB_CHUNK, SEQ), jnp.int32),
            pltpu.VMEM((NB_CHUNK, SEQ, D_MODEL), jnp.float32),
            pltpu.SemaphoreType.DMA,
        ],
    )
    def gather_scale(table_hbm, x_hbm, out_hbm, idx_v, rows_v, sem):
        wid = lax.axis_index("s") * NUM_CORES + lax.axis_index("c")
        b0 = wid * NB_PER_WORKER

        @pl.loop(0, N_CHUNKS)
        def _(c):
            bb = b0 + c * NB_CHUNK
            pltpu.sync_copy(x_hbm.at[pl.ds(bb, NB_CHUNK), :], idx_v)

            @pl.loop(0, NB_CHUNK)
            def _(q):
                for g0, j_lo, j_hi in GROUPS:
                    v = idx_v[q, pl.ds(g0, LANES)]
                    for j in range(j_lo, j_hi):
                        pltpu.async_copy(
                            table_hbm.at[pl.ds(v[j], 1), :],
                            rows_v.at[q, pl.ds(g0 + j, 1), :],
                            sem,
                        )

            @pl.loop(0, NB_CHUNK * SEQ)
            def _(r):
                pltpu.make_async_copy(
                    table_hbm.at[pl.ds(0, 1), :],
                    rows_v.at[0, pl.ds(0, 1), :],
                    sem,
                ).wait()

            @pl.loop(0, NB_CHUNK)
            def _(q):
                @pl.loop(0, SEQ)
                def _(s):
                    @pl.loop(0, D_MODEL, step=LANES)
                    def _(l):
                        slc = (q, pl.ds(s, 1), pl.ds(l, LANES))
                        rows_v.at[*slc][...] = rows_v.at[*slc][...] * SCALE

            pltpu.sync_copy(rows_v, out_hbm.at[pl.ds(bb, NB_CHUNK)])

    return gather_scale


_gather_scale = _make_gather_kernel()


@jax.jit
def kernel(x, table):
    return _gather_scale(table, x.astype(jnp.int32))
